# Initial kernel scaffold; baseline (speedup 1.0000x reference)
import jax
import jax.numpy as jnp
from jax.experimental import pallas as pl


def kernel(nNeigh, atom_i_idx, atom_j_idx, dist_ij, atoms_xyz, atoms_long, embed_table):
    raise NotImplementedError("write your pallas kernel here")


# speedup vs baseline: 1.9602x; 1.9602x over previous
import functools

import jax
import jax.numpy as jnp
from jax import lax
from jax.experimental import pallas as pl
from jax.experimental.pallas import tpu as pltpu
from jax.experimental.pallas import tpu_sc as plsc

N_CENTER = 50000
N_NEIGH = 4
F = 32
OUTW = 3 + 3 * F
PAIRS = [(j, k) for j in range(N_NEIGH) for k in range(N_NEIGH) if j != k]
NP_ = len(PAIRS)
ROW = NP_ * OUTW

NC, NS, L = 2, 16, 16
NW = NC * NS
G = L
N_GROUPS = N_CENTER // G
GPW_HI = -(-N_GROUPS // NW)
CPW = GPW_HI * G
N_LO = NW * (GPW_HI - 1)
REM = N_GROUPS - N_LO
GBUF = G * ROW


def _safe_sqrt(sq):
    i = plsc.bitcast(sq, jnp.int32)
    y = plsc.bitcast(jnp.int32(0x5F3759DF) - (i >> 1), jnp.float32)
    for _ in range(3):
        y = y * (1.5 - 0.5 * sq * y * y)
    return jnp.where(sq > 0.0, sq * y, 0.0)


def _tec_body(jflat, iidx, dflat, xs, ys, zs, types, emb, ang,
              emb_v, ji_v, ii_v, d_v, xv_v, yv_v, zv_v, tj_v, ti_v, out_v,
              sem_in, sem_out):
    wid = lax.axis_index("s") * NC + lax.axis_index("c")
    g0 = wid * (GPW_HI - 1) + jnp.minimum(wid, REM)
    ng = (GPW_HI - 1) + (wid < REM).astype(jnp.int32)
    base_c = g0 * G
    in_base = jnp.minimum(base_c, N_CENTER - CPW)
    off = base_c - in_base

    cps = [
        pltpu.async_copy(dflat.at[pl.ds(in_base * N_NEIGH, CPW * N_NEIGH)],
                         d_v, sem_in),
        pltpu.async_copy(jflat.at[pl.ds(in_base * N_NEIGH, CPW * N_NEIGH)],
                         ji_v, sem_in),
        pltpu.async_copy(iidx.at[pl.ds(in_base, CPW)], ii_v, sem_in),
        pltpu.async_copy(emb, emb_v, sem_in),
    ]
    for c in cps:
        c.wait()

    gcs = []
    NJ = CPW * N_NEIGH
    for c in range(NJ // 128):
        sl = pl.ds(c * 128, 128)
        gcs.append(pltpu.async_copy(xs.at[ji_v.at[sl]], xv_v.at[sl], sem_in))
        gcs.append(pltpu.async_copy(ys.at[ji_v.at[sl]], yv_v.at[sl], sem_in))
        gcs.append(pltpu.async_copy(zs.at[ji_v.at[sl]], zv_v.at[sl], sem_in))
        gcs.append(pltpu.async_copy(types.at[ji_v.at[sl]], tj_v.at[sl], sem_in))
    for c in range(CPW // 112):
        sl = pl.ds(c * 112, 112)
        gcs.append(pltpu.async_copy(types.at[ii_v.at[sl]], ti_v.at[sl], sem_in))
    for c in gcs:
        c.wait()

    lane = lax.iota(jnp.int32, L)
    one = jnp.full((L,), 1, jnp.int32)

    @pl.loop(0, ng)
    def _group(g):
        lc = lane + g * G + off
        lj = lc * N_NEIGH
        d = [plsc.load_gather(d_v, [lj + j]) for j in range(N_NEIGH)]
        tj = [plsc.load_gather(tj_v, [lj + j]) for j in range(N_NEIGH)]
        ti = plsc.load_gather(ti_v, [lc])
        x = [[plsc.load_gather(cv, [lj + j]) for cv in (xv_v, yv_v, zv_v)]
             for j in range(N_NEIGH)]
        ei = [plsc.load_gather(emb_v, [ti * F + f]) for f in range(F)]
        ejd = [[plsc.load_gather(emb_v, [tj[j] * F + f]) / d[j]
                for f in range(F)] for j in range(N_NEIGH)]

        buf = (g % 2) * GBUF
        obase = buf + lane * ROW
        tjk = {}
        for (j, k) in PAIRS:
            if j < k:
                dx = x[j][0] - x[k][0]
                dy = x[j][1] - x[k][1]
                dz = x[j][2] - x[k][2]
                sq = dx * dx + dy * dy + dz * dz
                tjk[(j, k)] = _safe_sqrt(sq)
            else:
                tjk[(j, k)] = tjk[(k, j)]

        for p, (j, k) in enumerate(PAIRS):
            c0 = obase + p * OUTW
            mind = jnp.minimum(d[j], d[k])
            maxd = jnp.maximum(d[j], d[k])
            tn = (tjk[(j, k)] - maxd + mind) / (2.0 * mind)
            plsc.store_scatter(out_v, [c0], d[j])
            plsc.store_scatter(out_v, [c0 + 1], d[k])
            plsc.store_scatter(out_v, [c0 + 2], tn)
            for f in range(F):
                plsc.store_scatter(out_v, [c0 + 3 + f], ei[f])
                plsc.store_scatter(out_v, [c0 + 3 + F + f], ejd[j][f])
                plsc.store_scatter(out_v, [c0 + 3 + 2 * F + f], ejd[k][f])

        pltpu.async_copy(
            out_v.at[pl.ds(buf, GBUF)],
            ang.at[pl.ds((base_c + g * G) * ROW, GBUF)],
            sem_out).wait()


@jax.jit
def _run(jflat, iidx, dflat, xs, ys, zs, types, emb_flat):
    mesh = plsc.VectorSubcoreMesh(core_axis_name="c", subcore_axis_name="s",
                                  num_cores=NC, num_subcores=NS)
    kern = functools.partial(
        pl.kernel,
        out_type=jax.ShapeDtypeStruct((N_CENTER * ROW,), jnp.float32),
        mesh=mesh,
        compiler_params=pltpu.CompilerParams(needs_layout_passes=False,
                                             use_tc_tiling_on_sc=False),
        scratch_types=[
            pltpu.VMEM((100 * F,), jnp.float32),
            pltpu.VMEM((CPW * N_NEIGH,), jnp.int32),
            pltpu.VMEM((CPW,), jnp.int32),
            pltpu.VMEM((CPW * N_NEIGH,), jnp.float32),
            pltpu.VMEM((CPW * N_NEIGH,), jnp.float32),
            pltpu.VMEM((CPW * N_NEIGH,), jnp.float32),
            pltpu.VMEM((CPW * N_NEIGH,), jnp.float32),
            pltpu.VMEM((CPW * N_NEIGH,), jnp.int32),
            pltpu.VMEM((CPW,), jnp.int32),
            pltpu.VMEM((2 * GBUF,), jnp.float32),
            pltpu.SemaphoreType.DMA,
            pltpu.SemaphoreType.DMA,
        ],
    )(_tec_body)
    return kern(jflat, iidx, dflat, xs, ys, zs, types, emb_flat)


def kernel(nNeigh, atom_i_idx, atom_j_idx, dist_ij, atoms_xyz, atoms_long,
           embed_table):
    jflat = atom_j_idx.reshape(-1).astype(jnp.int32)
    iidx = atom_i_idx.astype(jnp.int32)
    dflat = dist_ij.reshape(-1)
    emb_flat = embed_table.reshape(-1)
    types = atoms_long[:, 1].astype(jnp.int32)
    xs, ys, zs = (atoms_xyz[:, 0], atoms_xyz[:, 1], atoms_xyz[:, 2])
    ang = _run(jflat, iidx, dflat, xs, ys, zs, types, emb_flat)
    return atom_i_idx.reshape(-1), ang.reshape(N_CENTER, NP_, OUTW)
